# baseline (speedup 1.0000x reference)
import functools

import jax
import jax.numpy as jnp
from jax import lax
from jax.experimental import pallas as pl
from jax.experimental.pallas import tpu as pltpu
from jax.experimental.pallas import tpu_sc as plsc

_B, _D, _Q, _L = 32, 8, 15, 500
_V, _E, _NB = 100000, 300, 5
_EPB = 320
_NRW = 16
_NW = 32
_DPW = (_B * _D * _L) // _NW
_CH = 80
_NCH = _DPW // _CH
_QPW = 16
_NBUF = 4
_LOOKAHEAD = 2


@functools.cache
def _sc_gather_build():
    mesh = plsc.VectorSubcoreMesh(
        core_axis_name="c", subcore_axis_name="s", num_cores=2)

    @functools.partial(
        pl.kernel,
        mesh=mesh,
        out_type=(
            jax.ShapeDtypeStruct((_B * _D * _L, _EPB), jnp.bfloat16),
            jax.ShapeDtypeStruct((_B * _D * _L, _NRW), jnp.float32),
            jax.ShapeDtypeStruct((_B * _QPW, _EPB), jnp.bfloat16),
            jax.ShapeDtypeStruct((_B * _QPW, _NRW), jnp.float32),
        ),
        scratch_types=[
            pltpu.VMEM((_DPW,), jnp.int32),
            pltpu.VMEM((_QPW,), jnp.int32),
        ] + [pltpu.VMEM((_CH, _EPB), jnp.bfloat16) for _ in range(_NBUF)]
          + [pltpu.VMEM((_CH, _NRW), jnp.float32) for _ in range(_NBUF)]
          + [pltpu.SemaphoreType.DMA for _ in range(2 * _NBUF)],
        compiler_params=pltpu.CompilerParams(use_tc_tiling_on_sc=False),
    )
    def sc_gather(emb_hbm, nrm_hbm, didx_hbm, qidx_hbm,
                  dout_hbm, dn_hbm, qout_hbm, qn_hbm,
                  didx_v, qidx_v, *bufs_and_sems):
        ebufs = bufs_and_sems[:_NBUF]
        nbufs = bufs_and_sems[_NBUF:2 * _NBUF]
        gsem = bufs_and_sems[2 * _NBUF:3 * _NBUF]
        wsem = bufs_and_sems[3 * _NBUF:]
        wid = lax.axis_index("s") * 2 + lax.axis_index("c")
        dbase = wid * _DPW
        qbase = wid * _QPW
        pltpu.sync_copy(didx_hbm.at[pl.ds(dbase, _DPW)], didx_v)
        pltpu.sync_copy(qidx_hbm.at[pl.ds(qbase, _QPW)], qidx_v)
        pltpu.async_copy(
            emb_hbm.at[qidx_v], ebufs[0].at[pl.ds(0, _QPW)], gsem[0]).wait()
        pltpu.async_copy(
            nrm_hbm.at[qidx_v], nbufs[0].at[pl.ds(0, _QPW)], gsem[0]).wait()
        pltpu.sync_copy(
            ebufs[0].at[pl.ds(0, _QPW)], qout_hbm.at[pl.ds(qbase, _QPW)])
        pltpu.sync_copy(
            nbufs[0].at[pl.ds(0, _QPW)], qn_hbm.at[pl.ds(qbase, _QPW)])

        def g_start(k):
            b = k % _NBUF
            idx = didx_v.at[pl.ds(k * _CH, _CH)]
            return (pltpu.async_copy(emb_hbm.at[idx], ebufs[b], gsem[b]),
                    pltpu.async_copy(nrm_hbm.at[idx], nbufs[b], gsem[b]))

        gh = [None] * _NCH
        wh = [None] * _NCH
        for k in range(_LOOKAHEAD):
            gh[k] = g_start(k)
        for k in range(_NCH):
            ka = k + _LOOKAHEAD
            if ka < _NCH:
                kw = ka - _NBUF
                if kw >= 0:
                    wh[kw][0].wait()
                    wh[kw][1].wait()
                gh[ka] = g_start(ka)
            gh[k][0].wait()
            gh[k][1].wait()
            b = k % _NBUF
            dst = pl.ds(dbase + k * _CH, _CH)
            wh[k] = (pltpu.async_copy(ebufs[b], dout_hbm.at[dst], wsem[b]),
                     pltpu.async_copy(nbufs[b], dn_hbm.at[dst], wsem[b]))
        for k in range(max(0, _NCH - _NBUF), _NCH):
            wh[k][0].wait()
            wh[k][1].wait()

    return sc_gather


_PADBLK = 2000


def _pad_body(x_ref, eb_ref, n_ref):
    x = x_ref[...]
    eb_ref[:, 0:_E] = x.astype(jnp.bfloat16)
    eb_ref[:, _E:_EPB] = jnp.zeros((_PADBLK, _EPB - _E), jnp.bfloat16)
    nrm = jnp.sqrt(jnp.sum(x * x, axis=1, keepdims=True))
    n_ref[...] = jnp.broadcast_to(nrm, (_PADBLK, _NRW))


def _pad_table(emb):
    return pl.pallas_call(
        _pad_body,
        grid=(_V // _PADBLK,),
        in_specs=[pl.BlockSpec((_PADBLK, _E), lambda i: (i, 0))],
        out_specs=[
            pl.BlockSpec((_PADBLK, _EPB), lambda i: (i, 0)),
            pl.BlockSpec((_PADBLK, _NRW), lambda i: (i, 0)),
        ],
        out_shape=[
            jax.ShapeDtypeStruct((_V, _EPB), jnp.bfloat16),
            jax.ShapeDtypeStruct((_V, _NRW), jnp.float32),
        ],
    )(emb)


def _tc_body(d_ref, dn_ref, q_ref, qn_ref, gw_ref, pp_ref, out_ref):
    d = d_ref[...]
    q = q_ref[0:_Q, :]
    dots = lax.dot_general(
        d, q, (((1,), (1,)), ((), ())),
        preferred_element_type=jnp.float32,
        precision=lax.Precision.DEFAULT)
    dn = dn_ref[:, 0:1]
    qn = qn_ref[0:_Q, 0][None, :]
    denom = jnp.maximum(dn * qn, 1e-8)
    cos = jnp.clip(dots / denom, -1.0, 1.0)
    def _r(x):
        return x.astype(jnp.bfloat16).astype(jnp.float32)

    glog = jnp.sum(q.astype(jnp.float32) * _r(gw_ref[...]),
                   axis=1) + pp_ref[0, 10]
    e = jnp.exp(glog - jnp.max(glog))
    tw = e / jnp.sum(e)
    scores = []
    for dd in range(_D):
        cs = cos[dd * _L:(dd + 1) * _L]
        cnt = [jnp.sum((cs >= t).astype(jnp.float32), axis=0)
               for t in (-0.5, 0.0, 0.5, 1.0)]
        h = [jnp.float32(_L) - cnt[0], cnt[0] - cnt[1], cnt[1] - cnt[2],
             cnt[2] - cnt[3], cnt[3]]
        hw = sum(_r(h[k]) * _r(pp_ref[0, k]) for k in range(5))
        ffnn = (_r(hw + pp_ref[0, 5]) * _r(pp_ref[0, 6])) + pp_ref[0, 7]
        s = jnp.sum(ffnn * tw)
        scores.append(_r(s) * _r(pp_ref[0, 8]) + pp_ref[0, 9])
    out_ref[...] = jnp.stack(scores).reshape(1, 1, _D)


def kernel(batch_queries, batch_docs, emb, gate_w, gate_b,
           ffnn_w1, ffnn_b1, ffnn_w2, ffnn_b2, out_w, out_b):
    embb, nrm = _pad_table(emb)
    didx = batch_docs.reshape(-1).astype(jnp.int32)
    qpad = jnp.zeros((_B, _QPW - _Q), jnp.int32)
    qidx = jnp.concatenate(
        [batch_queries.astype(jnp.int32), qpad], axis=1).reshape(-1)
    d_emb, d_nrm, q_emb, q_nrm = _sc_gather_build()(embb, nrm, didx, qidx)
    gw_row = jnp.pad(gate_w.reshape(1, _E), ((0, 0), (0, _EPB - _E)))
    pp = jnp.concatenate([
        ffnn_w1.reshape(-1), ffnn_b1.reshape(-1), ffnn_w2.reshape(-1),
        ffnn_b2.reshape(-1), out_w.reshape(-1), out_b.reshape(-1),
        gate_b.reshape(-1), jnp.zeros((5,), jnp.float32)]).reshape(1, 16)
    return pl.pallas_call(
        _tc_body,
        grid=(_B,),
        in_specs=[
            pl.BlockSpec((_D * _L, _EPB), lambda b: (b, 0)),
            pl.BlockSpec((_D * _L, _NRW), lambda b: (b, 0)),
            pl.BlockSpec((_QPW, _EPB), lambda b: (b, 0)),
            pl.BlockSpec((_QPW, _NRW), lambda b: (b, 0)),
            pl.BlockSpec((1, _EPB), lambda b: (0, 0)),
            pl.BlockSpec((1, 16), lambda b: (0, 0)),
        ],
        out_specs=pl.BlockSpec((1, 1, _D), lambda b: (b, 0, 0)),
        out_shape=jax.ShapeDtypeStruct((_B, 1, _D), jnp.float32),
    )(d_emb, d_nrm, q_emb, q_nrm, gw_row, pp).reshape(_B, _D)


# speedup vs baseline: 1.3205x; 1.3205x over previous
import functools

import jax
import jax.numpy as jnp
from jax import lax
from jax.experimental import pallas as pl
from jax.experimental.pallas import tpu as pltpu
from jax.experimental.pallas import tpu_sc as plsc

_B, _D, _Q, _L = 32, 8, 15, 500
_V, _E, _NB = 100000, 300, 5
_EP = 304
_NW = 32
_DPW = (_B * _D * _L) // _NW
_CH = 80
_NCH = _DPW // _CH
_QPW = 16


_NBUF = 4
_LOOKAHEAD = 2


@functools.cache
def _sc_gather_build():
    mesh = plsc.VectorSubcoreMesh(
        core_axis_name="c", subcore_axis_name="s", num_cores=2)

    @functools.partial(
        pl.kernel,
        mesh=mesh,
        out_type=(
            jax.ShapeDtypeStruct((_B * _D * _L, _EP), jnp.float32),
            jax.ShapeDtypeStruct((_B * _QPW, _EP), jnp.float32),
        ),
        scratch_types=[
            pltpu.VMEM((_DPW,), jnp.int32),
            pltpu.VMEM((_QPW,), jnp.int32),
        ] + [pltpu.VMEM((_CH, _EP), jnp.float32) for _ in range(_NBUF)]
          + [pltpu.SemaphoreType.DMA for _ in range(2 * _NBUF)],
        compiler_params=pltpu.CompilerParams(use_tc_tiling_on_sc=False),
    )
    def sc_gather(emb_hbm, didx_hbm, qidx_hbm, dout_hbm, qout_hbm,
                  didx_v, qidx_v, *bufs_and_sems):
        bufs = bufs_and_sems[:_NBUF]
        gsem = bufs_and_sems[_NBUF:2 * _NBUF]
        wsem = bufs_and_sems[2 * _NBUF:]
        wid = lax.axis_index("s") * 2 + lax.axis_index("c")
        dbase = wid * _DPW
        qbase = wid * _QPW
        pltpu.sync_copy(didx_hbm.at[pl.ds(dbase, _DPW)], didx_v)
        pltpu.sync_copy(qidx_hbm.at[pl.ds(qbase, _QPW)], qidx_v)
        pltpu.async_copy(
            emb_hbm.at[qidx_v], bufs[0].at[pl.ds(0, _QPW)], gsem[0]).wait()
        pltpu.sync_copy(
            bufs[0].at[pl.ds(0, _QPW)], qout_hbm.at[pl.ds(qbase, _QPW)])

        def g_start(k):
            b = k % _NBUF
            return pltpu.async_copy(
                emb_hbm.at[didx_v.at[pl.ds(k * _CH, _CH)]], bufs[b], gsem[b])

        gh = [None] * _NCH
        wh = [None] * _NCH
        for k in range(_LOOKAHEAD):
            gh[k] = g_start(k)
        for k in range(_NCH):
            ka = k + _LOOKAHEAD
            if ka < _NCH:
                kw = ka - _NBUF
                if kw >= 0:
                    wh[kw].wait()
                gh[ka] = g_start(ka)
            gh[k].wait()
            b = k % _NBUF
            wh[k] = pltpu.async_copy(
                bufs[b], dout_hbm.at[pl.ds(dbase + k * _CH, _CH)], wsem[b])
        for k in range(max(0, _NCH - _NBUF), _NCH):
            wh[k].wait()

    return sc_gather


_PADBLK = 2000


def _pad_body(x_ref, o_ref):
    o_ref[:, 0:_E] = x_ref[...]
    o_ref[:, _E:_EP] = jnp.zeros((_PADBLK, _EP - _E), jnp.float32)


def _pad_table(emb):
    return pl.pallas_call(
        _pad_body,
        grid=(_V // _PADBLK,),
        in_specs=[pl.BlockSpec((_PADBLK, _E), lambda i: (i, 0))],
        out_specs=pl.BlockSpec((_PADBLK, _EP), lambda i: (i, 0)),
        out_shape=jax.ShapeDtypeStruct((_V, _EP), jnp.float32),
    )(emb)


def _tc_body(d_ref, q_ref, gw_ref, pp_ref, out_ref):
    d = d_ref[...]
    q = q_ref[0:_Q, :]
    dots = lax.dot_general(
        d, q, (((1,), (1,)), ((), ())),
        preferred_element_type=jnp.float32,
        precision=lax.Precision.DEFAULT)
    dn = jnp.sqrt(jnp.sum(d * d, axis=1, keepdims=True))
    qn = jnp.sqrt(jnp.sum(q * q, axis=1))[None, :]
    denom = jnp.maximum(dn * qn, 1e-8)
    cos = jnp.clip(dots / denom, -1.0, 1.0)
    def _r(x):
        return x.astype(jnp.bfloat16).astype(jnp.float32)

    glog = jnp.sum(_r(q) * _r(gw_ref[...]), axis=1) + pp_ref[0, 10]
    e = jnp.exp(glog - jnp.max(glog))
    tw = e / jnp.sum(e)
    scores = []
    for dd in range(_D):
        cs = cos[dd * _L:(dd + 1) * _L]
        cnt = [jnp.sum((cs >= t).astype(jnp.float32), axis=0)
               for t in (-0.5, 0.0, 0.5, 1.0)]
        h = [jnp.float32(_L) - cnt[0], cnt[0] - cnt[1], cnt[1] - cnt[2],
             cnt[2] - cnt[3], cnt[3]]
        hw = sum(_r(h[k]) * _r(pp_ref[0, k]) for k in range(5))
        ffnn = (_r(hw + pp_ref[0, 5]) * _r(pp_ref[0, 6])) + pp_ref[0, 7]
        s = jnp.sum(ffnn * tw)
        scores.append(_r(s) * _r(pp_ref[0, 8]) + pp_ref[0, 9])
    out_ref[...] = jnp.stack(scores).reshape(1, 1, _D)


def kernel(batch_queries, batch_docs, emb, gate_w, gate_b,
           ffnn_w1, ffnn_b1, ffnn_w2, ffnn_b2, out_w, out_b):
    embp = _pad_table(emb)
    didx = batch_docs.reshape(-1).astype(jnp.int32)
    qpad = jnp.zeros((_B, _QPW - _Q), jnp.int32)
    qidx = jnp.concatenate(
        [batch_queries.astype(jnp.int32), qpad], axis=1).reshape(-1)
    d_emb, q_emb = _sc_gather_build()(embp, didx, qidx)
    gw_row = jnp.pad(gate_w.reshape(1, _E), ((0, 0), (0, _EP - _E)))
    pp = jnp.concatenate([
        ffnn_w1.reshape(-1), ffnn_b1.reshape(-1), ffnn_w2.reshape(-1),
        ffnn_b2.reshape(-1), out_w.reshape(-1), out_b.reshape(-1),
        gate_b.reshape(-1), jnp.zeros((5,), jnp.float32)]).reshape(1, 16)
    return pl.pallas_call(
        _tc_body,
        grid=(_B,),
        in_specs=[
            pl.BlockSpec((_D * _L, _EP), lambda b: (b, 0)),
            pl.BlockSpec((_QPW, _EP), lambda b: (b, 0)),
            pl.BlockSpec((1, _EP), lambda b: (0, 0)),
            pl.BlockSpec((1, 16), lambda b: (0, 0)),
        ],
        out_specs=pl.BlockSpec((1, 1, _D), lambda b: (b, 0, 0)),
        out_shape=jax.ShapeDtypeStruct((_B, 1, _D), jnp.float32),
    )(d_emb, q_emb, gw_row, pp).reshape(_B, _D)
